# baseline (speedup 1.0000x reference)
import jax
import jax.numpy as jnp
from jax import lax
from jax.experimental import pallas as pl
from jax.experimental.pallas import tpu as pltpu
from jax.experimental.pallas import tpu_sc as plsc


_HW = 224 * 224
_FB = _HW // 8
_NCORE = 2
_NSUB = 16
_NW = _NCORE * _NSUB


def _sc_body(x_hbm, w_hbm, idx_hbm, out_hbm,
             idx_v, rows_v, fbuf, rowbufs, gsem, insems, outsems, fillsem):
    wid = lax.axis_index("c") * _NSUB + lax.axis_index("s")

    pltpu.sync_copy(idx_hbm, idx_v)
    pltpu.sync_copy(w_hbm, rows_v)
    fvec = plsc.load_gather(rows_v, [idx_v[...],
                                     jnp.full((16,), wid, jnp.int32)])

    def _fill_store(i, carry):
        fbuf[pl.ds(i * 16, 16)] = fvec
        return carry

    lax.fori_loop(0, _FB // 16, _fill_store, 0)

    fill_handles = []
    for b in range(4):
        row_off = (b * 128 + 96) * _HW + wid * _HW
        for j in range(8):
            fill_handles.append(pltpu.async_copy(
                fbuf, out_hbm.at[pl.ds(row_off + j * _FB, _FB)], fillsem))

    copy_rows = [(t // 4, 32 * (t % 4)) for t in range(16) if t % 4 != 3]
    in_h = [None] * len(copy_rows)
    out_h = [None] * len(copy_rows)
    for i, (b, cbase) in enumerate(copy_rows):
        slot = i % 2
        x_off = (b * 96 + cbase) * _HW + wid * _HW
        o_off = (b * 128 + cbase) * _HW + wid * _HW
        if i >= 2:
            out_h[i - 2].wait()
        in_h[i] = pltpu.async_copy(
            x_hbm.at[pl.ds(x_off, _HW)], rowbufs.at[slot], insems.at[slot])
        if i >= 1:
            pb, pc = copy_rows[i - 1]
            in_h[i - 1].wait()
            out_h[i - 1] = pltpu.async_copy(
                rowbufs.at[(i - 1) % 2],
                out_hbm.at[pl.ds((pb * 128 + pc) * _HW + wid * _HW, _HW)],
                outsems.at[(i - 1) % 2])
    last = len(copy_rows) - 1
    lb, lc = copy_rows[last]
    in_h[last].wait()
    out_h[last] = pltpu.async_copy(
        rowbufs.at[last % 2],
        out_hbm.at[pl.ds((lb * 128 + lc) * _HW + wid * _HW, _HW)],
        outsems.at[last % 2])
    out_h[last - 1].wait()
    out_h[last].wait()
    for h in fill_handles:
        h.wait()


def kernel(x, W, view_idx):
    B, C, H, Wd = x.shape
    hw = H * Wd
    x_flat = x.reshape(B * C * hw)
    idx16 = jnp.full((16,), view_idx, jnp.int32)

    mesh = plsc.VectorSubcoreMesh(core_axis_name="c", subcore_axis_name="s")
    out_flat = pl.kernel(
        _sc_body,
        out_type=jax.ShapeDtypeStruct((B * 128 * hw,), x.dtype),
        mesh=mesh,
        compiler_params=pltpu.CompilerParams(needs_layout_passes=False),
        scratch_types=[
            pltpu.VMEM((16,), jnp.int32),
            pltpu.VMEM((100, 32), jnp.float32),
            pltpu.VMEM((_FB,), jnp.float32),
            pltpu.VMEM((2, _HW), jnp.float32),
            pltpu.SemaphoreType.DMA,
            pltpu.SemaphoreType.DMA((2,)),
            pltpu.SemaphoreType.DMA((2,)),
            pltpu.SemaphoreType.DMA,
        ],
    )(x_flat, W, idx16)
    return out_flat.reshape(B, 128, H, Wd)
